# baseline (speedup 1.0000x reference)
import functools

import jax
import jax.numpy as jnp
from jax import lax
from jax.experimental import pallas as pl
from jax.experimental.pallas import tpu as pltpu
from jax.experimental.pallas import tpu_sc as plsc

_N = 10000
_E = 320000
_H = 128
_HH = _H // 2

_NC = 2
_NS = 16
_EPT = _E // _NS
_G = 80
_R = _EPT // _G
_UNROLL = 4
_NCH = _HH // 16
_NPS = 624
_NTAIL = _N - _NS * _NPS

_BR = 2000
_GRID = _N // _BR


def _proj(h, w_ref, b):
    return jnp.dot(h, w_ref[...], preferred_element_type=jnp.float32) + b


def _kqvs(h, kw, kb, eb, qw, qb, vw, vb, sw, sb, k_out, qv_out, skip_out):
    k = _proj(h, kw, kb[...] + eb[...])
    q = _proj(h, qw, qb[...])
    v = _proj(h, vw, vb[...])
    k_out[0] = k[:, :_HH]
    k_out[1] = k[:, _HH:]
    qv_out[0] = jnp.concatenate([q[:, :_HH], v[:, :_HH]], axis=1)
    qv_out[1] = jnp.concatenate([q[:, _HH:], v[:, _HH:]], axis=1)
    skip_out[...] = _proj(h, sw, sb[...])


def _tc_first_body(x_ref, in_w, in_b, kw, kb, eb, qw, qb, vw, vb, sw, sb,
                   k_out, qv_out, skip_out):
    h = _proj(x_ref[...], in_w, in_b[...])
    _kqvs(h, kw, kb, eb, qw, qb, vw, vb, sw, sb, k_out, qv_out, skip_out)


def _ln_gelu(agg_ref, skip_ref, g, be):
    out = jnp.concatenate([agg_ref[0], agg_ref[1]], axis=1) + skip_ref[...]
    mu = jnp.mean(out, axis=-1, keepdims=True)
    var = jnp.mean((out - mu) ** 2, axis=-1, keepdims=True)
    xn = (out - mu) * lax.rsqrt(var + 1e-5) * g[...] + be[...]
    return 0.5 * xn * (1.0 + lax.erf(xn * (2.0 ** -0.5)))


def _tc_mid_body(agg_ref, skip_ref, g, be, kw, kb, eb, qw, qb, vw, vb, sw, sb,
                 k_out, qv_out, skip_out):
    h = _ln_gelu(agg_ref, skip_ref, g, be)
    _kqvs(h, kw, kb, eb, qw, qb, vw, vb, sw, sb, k_out, qv_out, skip_out)


def _tc_final_body(agg_ref, skip_ref, g, be, ow, y_out):
    h = _ln_gelu(agg_ref, skip_ref, g, be)
    y_out[...] = jnp.dot(h, ow[...], preferred_element_type=jnp.float32)


def _wspec(shape):
    nd = len(shape)
    return pl.BlockSpec(shape, lambda i, _n=nd: (0,) * _n)


_ROW = pl.BlockSpec((_BR, _H), lambda i: (i, 0))
_HALF = pl.BlockSpec((2, _BR, _HH), lambda i: (0, i, 0))
_HALF2 = pl.BlockSpec((2, _BR, _H), lambda i: (0, i, 0))

_KQVS_OUT = dict(
    out_specs=[_HALF, _HALF2, _ROW],
    out_shape=[
        jax.ShapeDtypeStruct((2, _N, _HH), jnp.float32),
        jax.ShapeDtypeStruct((2, _N, _H), jnp.float32),
        jax.ShapeDtypeStruct((_N, _H), jnp.float32),
    ],
)

_W = _wspec((_H, _H))
_B = _wspec((1, _H))


def _tc_first(x, in_w, in_b, blk):
    f = pl.pallas_call(
        _tc_first_body,
        grid=(_GRID,),
        in_specs=[_ROW, _W, _B] + [_W, _B, _B, _W, _B, _W, _B, _W, _B],
        **_KQVS_OUT,
    )
    return f(x, in_w, in_b.reshape(1, -1),
             blk['kw'], blk['kb'].reshape(1, -1), blk['eb'].reshape(1, -1),
             blk['qw'], blk['qb'].reshape(1, -1),
             blk['vw'], blk['vb'].reshape(1, -1),
             blk['sw'], blk['bias'].reshape(1, -1))


def _tc_mid(agg, skip, g, be, blk):
    f = pl.pallas_call(
        _tc_mid_body,
        grid=(_GRID,),
        in_specs=[_HALF, _ROW, _B, _B] + [_W, _B, _B, _W, _B, _W, _B, _W, _B],
        **_KQVS_OUT,
    )
    return f(agg, skip, g.reshape(1, -1), be.reshape(1, -1),
             blk['kw'], blk['kb'].reshape(1, -1), blk['eb'].reshape(1, -1),
             blk['qw'], blk['qb'].reshape(1, -1),
             blk['vw'], blk['vb'].reshape(1, -1),
             blk['sw'], blk['bias'].reshape(1, -1))


def _tc_final(agg, skip, g, be, ow):
    f = pl.pallas_call(
        _tc_final_body,
        grid=(_GRID,),
        in_specs=[_HALF, _ROW, _B, _B, _W],
        out_specs=_ROW,
        out_shape=jax.ShapeDtypeStruct((_N, _H), jnp.float32),
    )
    return f(agg, skip, g.reshape(1, -1), be.reshape(1, -1), ow)


def _sc_edge_body(k_hbm, qv_hbm, src_hbm, dst_hbm, ea_hbm, ew_hbm, zero_hbm,
                  out_hbm, dstall, srcb, eab, ewb, kbuf, qvbuf, msgb, agg,
                  semg0, semg1, sems0, sems1, semi0, semi1):
    semg = (semg0, semg1)
    sems = (sems0, sems1)
    semi = (semi0, semi1)
    cid = lax.axis_index("c")
    sid = lax.axis_index("s")
    row0 = sid * _NPS
    kh = k_hbm.at[cid]
    qvh = qv_hbm.at[cid]
    pltpu.sync_copy(zero_hbm.at[pl.ds(row0, _NPS)], agg.at[pl.ds(row0, _NPS)])
    @pl.when(sid == 0)
    def _init_tail():
        pltpu.sync_copy(zero_hbm.at[pl.ds(_NS * _NPS, _NTAIL)],
                        agg.at[pl.ds(_NS * _NPS, _NTAIL)])
    pltpu.sync_copy(ew_hbm.at[cid], ewb)
    pltpu.sync_copy(dst_hbm.at[sid], dstall)
    plsc.subcore_barrier()

    ewc = [ewb[pl.ds(c * 16, 16)] for c in range(_NCH)]

    def issue_idx(r, b):
        pltpu.async_copy(src_hbm.at[sid, r], srcb.at[b], semi[b])
        pltpu.async_copy(ea_hbm.at[sid, r], eab.at[b], semi[b])

    def wait_idx(b):
        pltpu.make_async_copy(src_hbm.at[sid, 0], srcb.at[b], semi[b]).wait()
        pltpu.make_async_copy(ea_hbm.at[sid, 0], eab.at[b], semi[b]).wait()

    def issue_gather(r, b):
        pltpu.async_copy(kh.at[dstall.at[r]], kbuf.at[b], semg[b])
        pltpu.async_copy(qvh.at[srcb.at[b]], qvbuf.at[b], semg[b])

    def wait_gather(b):
        pltpu.make_async_copy(kh.at[dstall.at[0]], kbuf.at[b],
                              semg[b]).wait()
        pltpu.make_async_copy(qvh.at[srcb.at[b]], qvbuf.at[b],
                              semg[b]).wait()

    def wait_scatter(b):
        pltpu.make_async_copy(msgb.at[b], agg.at[dstall.at[0]],
                              sems[b]).wait()

    def do_round(r, b):
        wait_gather(b)
        @pl.when(r + 1 < _R)
        def _prefetch():
            wait_idx(1 - b)
            issue_gather(r + 1, 1 - b)
        @pl.when(r >= 1)
        def _drain():
            wait_scatter(1 - b)

        @plsc.parallel_loop(0, _G, step=_UNROLL, unroll=4)
        def edge_block(i):
            for jj in range(_UNROLL):
                j = i + jj
                eaj = eab[b, j]
                for c in range(_NCH):
                    sl = pl.ds(c * 16, 16)
                    t = kbuf[b, j, sl] + qvbuf[b, j, sl] + eaj * ewc[c]
                    gate = 1.0 / (1.0 + jnp.exp(-t))
                    msgb[b, j, sl] = gate * qvbuf[b, j,
                                                  pl.ds(_HH + c * 16, 16)]

        pltpu.async_copy(msgb.at[b], agg.at[dstall.at[r]], sems[b], add=True)
        @pl.when(r + 2 < _R)
        def _idx_prefetch():
            issue_idx(r + 2, b)

    issue_idx(0, 0)
    wait_idx(0)
    issue_gather(0, 0)
    issue_idx(1, 1)

    def pair_body(rr, carry):
        do_round(rr * 2, 0)
        do_round(rr * 2 + 1, 1)
        return carry

    lax.fori_loop(0, _R // 2, pair_body, 0)
    if _R % 2:
        do_round(_R - 1, 0)
        wait_scatter(0)
    else:
        wait_scatter(1)
    plsc.subcore_barrier()
    pltpu.sync_copy(agg.at[pl.ds(row0, _NPS)],
                    out_hbm.at[cid, pl.ds(row0, _NPS)])
    @pl.when(sid == 0)
    def _out_tail():
        pltpu.sync_copy(agg.at[pl.ds(_NS * _NPS, _NTAIL)],
                        out_hbm.at[cid, pl.ds(_NS * _NPS, _NTAIL)])


@functools.lru_cache(maxsize=1)
def _sc_edge_kernel():
    return pl.kernel(
        _sc_edge_body,
        mesh=plsc.VectorSubcoreMesh(core_axis_name="c", subcore_axis_name="s"),
        compiler_params=pltpu.CompilerParams(use_tc_tiling_on_sc=False),
        out_type=jax.ShapeDtypeStruct((2, _N, _HH), jnp.float32),
        scratch_types=[
            pltpu.VMEM((_R, _G), jnp.int32),
            pltpu.VMEM((2, _G), jnp.int32),
            pltpu.VMEM((2, _G, 16), jnp.float32),
            pltpu.VMEM((_HH,), jnp.float32),
            pltpu.VMEM((2, _G, _HH), jnp.float32),
            pltpu.VMEM((2, _G, 2 * _HH), jnp.float32),
            pltpu.VMEM((2, _G, _HH), jnp.float32),
            pltpu.VMEM_SHARED((_N, _HH), jnp.float32),
            pltpu.SemaphoreType.DMA,
            pltpu.SemaphoreType.DMA,
            pltpu.SemaphoreType.DMA,
            pltpu.SemaphoreType.DMA,
            pltpu.SemaphoreType.DMA,
            pltpu.SemaphoreType.DMA,
        ],
    )


def _sc_edge(*args):
    return _sc_edge_kernel()(*args)


def kernel(x, edge_index, edge_attr, params):
    src = edge_index[0].reshape(_NS, _R, _G)
    dst = edge_index[1].reshape(_NS, _R, _G)
    edge_attr = jnp.broadcast_to(
        edge_attr.reshape(_NS, _R, _G, 1), (_NS, _R, _G, 16))
    zeros = jnp.zeros((_N, _HH), jnp.float32)
    blocks = params['blocks']
    k, qv, skip = _tc_first(x, params['in_w'], params['in_b'], blocks[0])
    y = None
    for i in range(len(blocks)):
        blk = blocks[i]
        ew2 = blk['ew'][0].reshape(2, _HH)
        agg = _sc_edge(k, qv, src, dst, edge_attr, ew2, zeros)
        if i + 1 < len(blocks):
            k, qv, skip = _tc_mid(agg, skip, blk['g'], blk['be'],
                                  blocks[i + 1])
        else:
            y = _tc_final(agg, skip, blk['g'], blk['be'], params['out_w'])
    return y


# speedup vs baseline: 2.1135x; 2.1135x over previous
import functools

import jax
import jax.numpy as jnp
from jax import lax
from jax.experimental import pallas as pl
from jax.experimental.pallas import tpu as pltpu
from jax.experimental.pallas import tpu_sc as plsc

_N = 10000
_E = 320000
_H = 128
_HH = _H // 2

_NC = 2
_NS = 16
_EPT = _E // _NS
_G = 80
_R = _EPT // _G
_UNROLL = 4
_NCH = _HH // 16
_NPS = 624
_NTAIL = _N - _NS * _NPS

_BR = 2000
_GRID = _N // _BR


def _proj(h, w_ref, b):
    return jnp.dot(h, w_ref[...], preferred_element_type=jnp.float32) + b


def _kqvs(h, kw, kb, eb, qw, qb, vw, vb, sw, sb, k_out, qv_out, skip_out):
    k = _proj(h, kw, kb[...] + eb[...])
    q = _proj(h, qw, qb[...])
    v = _proj(h, vw, vb[...])
    k_out[0] = k[:, :_HH]
    k_out[1] = k[:, _HH:]
    qv_out[0] = jnp.concatenate([q[:, :_HH], v[:, :_HH]], axis=1)
    qv_out[1] = jnp.concatenate([q[:, _HH:], v[:, _HH:]], axis=1)
    skip_out[...] = _proj(h, sw, sb[...])


def _tc_first_body(x_ref, in_w, in_b, kw, kb, eb, qw, qb, vw, vb, sw, sb,
                   k_out, qv_out, skip_out):
    h = _proj(x_ref[...], in_w, in_b[...])
    _kqvs(h, kw, kb, eb, qw, qb, vw, vb, sw, sb, k_out, qv_out, skip_out)


def _ln_gelu(agg_ref, skip_ref, g, be):
    out = jnp.concatenate([agg_ref[0], agg_ref[1]], axis=1) + skip_ref[...]
    mu = jnp.mean(out, axis=-1, keepdims=True)
    var = jnp.mean((out - mu) ** 2, axis=-1, keepdims=True)
    xn = (out - mu) * lax.rsqrt(var + 1e-5) * g[...] + be[...]
    return 0.5 * xn * (1.0 + lax.erf(xn * (2.0 ** -0.5)))


def _tc_mid_body(agg_ref, skip_ref, g, be, kw, kb, eb, qw, qb, vw, vb, sw, sb,
                 k_out, qv_out, skip_out):
    h = _ln_gelu(agg_ref, skip_ref, g, be)
    _kqvs(h, kw, kb, eb, qw, qb, vw, vb, sw, sb, k_out, qv_out, skip_out)


def _tc_final_body(agg_ref, skip_ref, g, be, ow, y_out):
    h = _ln_gelu(agg_ref, skip_ref, g, be)
    y_out[...] = jnp.dot(h, ow[...], preferred_element_type=jnp.float32)


def _wspec(shape):
    nd = len(shape)
    return pl.BlockSpec(shape, lambda i, _n=nd: (0,) * _n)


_ROW = pl.BlockSpec((_BR, _H), lambda i: (i, 0))
_HALF = pl.BlockSpec((2, _BR, _HH), lambda i: (0, i, 0))
_HALF2 = pl.BlockSpec((2, _BR, _H), lambda i: (0, i, 0))

_KQVS_OUT = dict(
    out_specs=[_HALF, _HALF2, _ROW],
    out_shape=[
        jax.ShapeDtypeStruct((2, _N, _HH), jnp.float32),
        jax.ShapeDtypeStruct((2, _N, _H), jnp.float32),
        jax.ShapeDtypeStruct((_N, _H), jnp.float32),
    ],
)

_W = _wspec((_H, _H))
_B = _wspec((1, _H))


def _tc_first(x, in_w, in_b, blk):
    f = pl.pallas_call(
        _tc_first_body,
        grid=(_GRID,),
        in_specs=[_ROW, _W, _B] + [_W, _B, _B, _W, _B, _W, _B, _W, _B],
        **_KQVS_OUT,
    )
    return f(x, in_w, in_b.reshape(1, -1),
             blk['kw'], blk['kb'].reshape(1, -1), blk['eb'].reshape(1, -1),
             blk['qw'], blk['qb'].reshape(1, -1),
             blk['vw'], blk['vb'].reshape(1, -1),
             blk['sw'], blk['bias'].reshape(1, -1))


def _tc_mid(agg, skip, g, be, blk):
    f = pl.pallas_call(
        _tc_mid_body,
        grid=(_GRID,),
        in_specs=[_HALF, _ROW, _B, _B] + [_W, _B, _B, _W, _B, _W, _B, _W, _B],
        **_KQVS_OUT,
    )
    return f(agg, skip, g.reshape(1, -1), be.reshape(1, -1),
             blk['kw'], blk['kb'].reshape(1, -1), blk['eb'].reshape(1, -1),
             blk['qw'], blk['qb'].reshape(1, -1),
             blk['vw'], blk['vb'].reshape(1, -1),
             blk['sw'], blk['bias'].reshape(1, -1))


def _tc_final(agg, skip, g, be, ow):
    f = pl.pallas_call(
        _tc_final_body,
        grid=(_GRID,),
        in_specs=[_HALF, _ROW, _B, _B, _W],
        out_specs=_ROW,
        out_shape=jax.ShapeDtypeStruct((_N, _H), jnp.float32),
    )
    return f(agg, skip, g.reshape(1, -1), be.reshape(1, -1), ow)


def _sc_edge_body(k_hbm, qv_hbm, src_hbm, dst_hbm, ea_hbm, ew_hbm, zero_hbm,
                  out_hbm, dstall, srcb, eab, ewb, kbuf, qvbuf, msgb, agg,
                  semg0, semg1, sems0, sems1, semi0, semi1):
    semg = (semg0, semg1)
    sems = (sems0, sems1)
    semi = (semi0, semi1)
    cid = lax.axis_index("c")
    sid = lax.axis_index("s")
    row0 = sid * _NPS
    kh = k_hbm.at[cid]
    qvh = qv_hbm.at[cid]
    pltpu.sync_copy(zero_hbm.at[pl.ds(row0, _NPS)], agg.at[pl.ds(row0, _NPS)])
    @pl.when(sid == 0)
    def _init_tail():
        pltpu.sync_copy(zero_hbm.at[pl.ds(_NS * _NPS, _NTAIL)],
                        agg.at[pl.ds(_NS * _NPS, _NTAIL)])
    pltpu.sync_copy(ew_hbm.at[cid], ewb)
    pltpu.sync_copy(dst_hbm.at[sid], dstall)
    plsc.subcore_barrier()

    ewc = [ewb[pl.ds(c * 16, 16)] for c in range(_NCH)]

    def issue_idx(r, b):
        pltpu.async_copy(src_hbm.at[sid, r], srcb.at[b], semi[b])
        pltpu.async_copy(ea_hbm.at[sid, r], eab.at[b], semi[b])

    def wait_idx(b):
        pltpu.make_async_copy(src_hbm.at[sid, 0], srcb.at[b], semi[b]).wait()
        pltpu.make_async_copy(ea_hbm.at[sid, 0], eab.at[b], semi[b]).wait()

    def issue_gather(r, b):
        pltpu.async_copy(kh.at[dstall.at[r]], kbuf.at[b], semg[b])
        pltpu.async_copy(qvh.at[srcb.at[b]], qvbuf.at[b], semg[b])

    def wait_gather(b):
        pltpu.make_async_copy(kh.at[dstall.at[0]], kbuf.at[b],
                              semg[b]).wait()
        pltpu.make_async_copy(qvh.at[srcb.at[b]], qvbuf.at[b],
                              semg[b]).wait()

    def wait_scatter(b):
        pltpu.make_async_copy(msgb.at[b], agg.at[dstall.at[0]],
                              sems[b]).wait()

    def do_round(r, b):
        wait_gather(b)
        @pl.when(r + 1 < _R)
        def _prefetch():
            wait_idx(1 - b)
            issue_gather(r + 1, 1 - b)
        @pl.when(r >= 1)
        def _drain():
            wait_scatter(1 - b)

        @plsc.parallel_loop(0, _G, step=_UNROLL, unroll=2)
        def edge_block(i):
            for jj in range(_UNROLL):
                j = i + jj
                eaj = eab[b, j]
                for c in range(_NCH):
                    sl = pl.ds(c * 16, 16)
                    t = kbuf[b, j, sl] + qvbuf[b, j, sl] + eaj * ewc[c]
                    gate = 1.0 / (1.0 + jnp.exp(-t))
                    msgb[b, j, sl] = gate * qvbuf[b, j,
                                                  pl.ds(_HH + c * 16, 16)]

        pltpu.async_copy(msgb.at[b], agg.at[dstall.at[r]], sems[b], add=True)
        @pl.when(r + 2 < _R)
        def _idx_prefetch():
            issue_idx(r + 2, b)

    issue_idx(0, 0)
    wait_idx(0)
    issue_gather(0, 0)
    issue_idx(1, 1)

    def pair_body(rr, carry):
        do_round(rr * 2, 0)
        do_round(rr * 2 + 1, 1)
        return carry

    lax.fori_loop(0, _R // 2, pair_body, 0)
    if _R % 2:
        do_round(_R - 1, 0)
        wait_scatter(0)
    else:
        wait_scatter(1)
    plsc.subcore_barrier()
    pltpu.sync_copy(agg.at[pl.ds(row0, _NPS)],
                    out_hbm.at[cid, pl.ds(row0, _NPS)])
    @pl.when(sid == 0)
    def _out_tail():
        pltpu.sync_copy(agg.at[pl.ds(_NS * _NPS, _NTAIL)],
                        out_hbm.at[cid, pl.ds(_NS * _NPS, _NTAIL)])


@functools.lru_cache(maxsize=1)
def _sc_edge_kernel():
    return pl.kernel(
        _sc_edge_body,
        mesh=plsc.VectorSubcoreMesh(core_axis_name="c", subcore_axis_name="s"),
        compiler_params=pltpu.CompilerParams(use_tc_tiling_on_sc=False),
        out_type=jax.ShapeDtypeStruct((2, _N, _HH), jnp.float32),
        scratch_types=[
            pltpu.VMEM((_R, _G), jnp.int32),
            pltpu.VMEM((2, _G), jnp.int32),
            pltpu.VMEM((2, _G, 16), jnp.float32),
            pltpu.VMEM((_HH,), jnp.float32),
            pltpu.VMEM((2, _G, _HH), jnp.float32),
            pltpu.VMEM((2, _G, 2 * _HH), jnp.float32),
            pltpu.VMEM((2, _G, _HH), jnp.float32),
            pltpu.VMEM_SHARED((_N, _HH), jnp.float32),
            pltpu.SemaphoreType.DMA,
            pltpu.SemaphoreType.DMA,
            pltpu.SemaphoreType.DMA,
            pltpu.SemaphoreType.DMA,
            pltpu.SemaphoreType.DMA,
            pltpu.SemaphoreType.DMA,
        ],
    )


def _sc_edge(*args):
    return _sc_edge_kernel()(*args)


def kernel(x, edge_index, edge_attr, params):
    src = edge_index[0].reshape(_NS, _R, _G)
    dst = edge_index[1].reshape(_NS, _R, _G)
    edge_attr = jnp.broadcast_to(
        edge_attr.reshape(_NS, _R, _G, 1), (_NS, _R, _G, 16))
    zeros = jnp.zeros((_N, _HH), jnp.float32)
    blocks = params['blocks']
    k, qv, skip = _tc_first(x, params['in_w'], params['in_b'], blocks[0])
    y = None
    for i in range(len(blocks)):
        blk = blocks[i]
        ew2 = blk['ew'][0].reshape(2, _HH)
        agg = _sc_edge(k, qv, src, dst, edge_attr, ew2, zeros)
        if i + 1 < len(blocks):
            k, qv, skip = _tc_mid(agg, skip, blk['g'], blk['be'],
                                  blocks[i + 1])
        else:
            y = _tc_final(agg, skip, blk['g'], blk['be'], params['out_w'])
    return y


# speedup vs baseline: 2.6485x; 1.2531x over previous
import functools

import numpy as np

import jax
import jax.numpy as jnp
from jax import lax
from jax.experimental import pallas as pl
from jax.experimental.pallas import tpu as pltpu
from jax.experimental.pallas import tpu_sc as plsc

_N = 10000
_E = 320000
_H = 128
_HH = _H // 2

_NC = 2
_NS = 16
_EPT = _E // _NS
_G = 80
_R = _EPT // _G
_UNROLL = 4
_NCH = _HH // 16
_NPS = 624
_NTAIL = _N - _NS * _NPS

_BR = 2000
_GRID = _N // _BR


def _proj(h, w_ref, b):
    return jnp.dot(h, w_ref[...], preferred_element_type=jnp.float32) + b


def _kqvs(h, kw, kb, eb, qw, qb, vw, vb, sw, sb, k_out, qv_out, skip_out):
    k = _proj(h, kw, kb[...] + eb[...]).astype(jnp.bfloat16)
    q = _proj(h, qw, qb[...]).astype(jnp.bfloat16)
    v = _proj(h, vw, vb[...]).astype(jnp.bfloat16)
    k_out[0] = k[:, :_HH]
    k_out[1] = k[:, _HH:]
    qv_out[0] = jnp.concatenate([q[:, :_HH], v[:, :_HH]], axis=1)
    qv_out[1] = jnp.concatenate([q[:, _HH:], v[:, _HH:]], axis=1)
    skip_out[...] = _proj(h, sw, sb[...])


def _tc_first_body(x_ref, in_w, in_b, kw, kb, eb, qw, qb, vw, vb, sw, sb,
                   k_out, qv_out, skip_out):
    h = _proj(x_ref[...], in_w, in_b[...])
    _kqvs(h, kw, kb, eb, qw, qb, vw, vb, sw, sb, k_out, qv_out, skip_out)


def _ln_gelu(agg_ref, skip_ref, g, be):
    out = jnp.concatenate([agg_ref[0], agg_ref[1]], axis=1) + skip_ref[...]
    mu = jnp.mean(out, axis=-1, keepdims=True)
    var = jnp.mean((out - mu) ** 2, axis=-1, keepdims=True)
    xn = (out - mu) * lax.rsqrt(var + 1e-5) * g[...] + be[...]
    return 0.5 * xn * (1.0 + lax.erf(xn * (2.0 ** -0.5)))


def _tc_mid_body(agg_ref, skip_ref, g, be, kw, kb, eb, qw, qb, vw, vb, sw, sb,
                 k_out, qv_out, skip_out):
    h = _ln_gelu(agg_ref, skip_ref, g, be)
    _kqvs(h, kw, kb, eb, qw, qb, vw, vb, sw, sb, k_out, qv_out, skip_out)


def _tc_final_body(agg_ref, skip_ref, g, be, ow, y_out):
    h = _ln_gelu(agg_ref, skip_ref, g, be)
    y_out[...] = jnp.dot(h, ow[...], preferred_element_type=jnp.float32)


def _wspec(shape):
    nd = len(shape)
    return pl.BlockSpec(shape, lambda i, _n=nd: (0,) * _n)


_ROW = pl.BlockSpec((_BR, _H), lambda i: (i, 0))
_HALF = pl.BlockSpec((2, _BR, _HH), lambda i: (0, i, 0))
_HALF2 = pl.BlockSpec((2, _BR, _H), lambda i: (0, i, 0))

_KQVS_OUT = dict(
    out_specs=[_HALF, _HALF2, _ROW],
    out_shape=[
        jax.ShapeDtypeStruct((2, _N, _HH), jnp.bfloat16),
        jax.ShapeDtypeStruct((2, _N, _H), jnp.bfloat16),
        jax.ShapeDtypeStruct((_N, _H), jnp.float32),
    ],
)

_W = _wspec((_H, _H))
_B = _wspec((1, _H))


def _tc_first(x, in_w, in_b, blk):
    f = pl.pallas_call(
        _tc_first_body,
        grid=(_GRID,),
        in_specs=[_ROW, _W, _B] + [_W, _B, _B, _W, _B, _W, _B, _W, _B],
        **_KQVS_OUT,
    )
    return f(x, in_w, in_b.reshape(1, -1),
             blk['kw'], blk['kb'].reshape(1, -1), blk['eb'].reshape(1, -1),
             blk['qw'], blk['qb'].reshape(1, -1),
             blk['vw'], blk['vb'].reshape(1, -1),
             blk['sw'], blk['bias'].reshape(1, -1))


def _tc_mid(agg, skip, g, be, blk):
    f = pl.pallas_call(
        _tc_mid_body,
        grid=(_GRID,),
        in_specs=[_HALF, _ROW, _B, _B] + [_W, _B, _B, _W, _B, _W, _B, _W, _B],
        **_KQVS_OUT,
    )
    return f(agg, skip, g.reshape(1, -1), be.reshape(1, -1),
             blk['kw'], blk['kb'].reshape(1, -1), blk['eb'].reshape(1, -1),
             blk['qw'], blk['qb'].reshape(1, -1),
             blk['vw'], blk['vb'].reshape(1, -1),
             blk['sw'], blk['bias'].reshape(1, -1))


def _tc_final(agg, skip, g, be, ow):
    f = pl.pallas_call(
        _tc_final_body,
        grid=(_GRID,),
        in_specs=[_HALF, _ROW, _B, _B, _W],
        out_specs=_ROW,
        out_shape=jax.ShapeDtypeStruct((_N, _H), jnp.float32),
    )
    return f(agg, skip, g.reshape(1, -1), be.reshape(1, -1), ow)


def _sc_edge_body(k_hbm, qv_hbm, src_hbm, dst_hbm, ea_hbm, ew_hbm, zero_hbm,
                  out_hbm, dstall, srcb, eab, ewb, kbuf, qvbuf, msgb, agg,
                  semg0, semg1, sems0, sems1, semi0, semi1):
    semg = (semg0, semg1)
    sems = (sems0, sems1)
    semi = (semi0, semi1)
    cid = lax.axis_index("c")
    sid = lax.axis_index("s")
    row0 = sid * _NPS
    kh = k_hbm.at[cid]
    qvh = qv_hbm.at[cid]
    pltpu.sync_copy(zero_hbm.at[pl.ds(row0, _NPS)], agg.at[pl.ds(row0, _NPS)])
    @pl.when(sid == 0)
    def _init_tail():
        pltpu.sync_copy(zero_hbm.at[pl.ds(_NS * _NPS, _NTAIL)],
                        agg.at[pl.ds(_NS * _NPS, _NTAIL)])
    pltpu.sync_copy(ew_hbm.at[cid], ewb)
    pltpu.sync_copy(dst_hbm.at[sid], dstall)
    plsc.subcore_barrier()

    ewc = [ewb[pl.ds(c * 16, 16)] for c in range(_NCH)]

    def issue_idx(r, b):
        pltpu.async_copy(src_hbm.at[sid, r], srcb.at[b], semi[b])
        pltpu.async_copy(ea_hbm.at[sid, r], eab.at[b], semi[b])

    def wait_idx(b):
        pltpu.make_async_copy(src_hbm.at[sid, 0], srcb.at[b], semi[b]).wait()
        pltpu.make_async_copy(ea_hbm.at[sid, 0], eab.at[b], semi[b]).wait()

    def issue_gather(r, b):
        pltpu.async_copy(kh.at[dstall.at[r]], kbuf.at[b], semg[b])
        pltpu.async_copy(qvh.at[srcb.at[b]], qvbuf.at[b], semg[b])

    def wait_gather(b):
        pltpu.make_async_copy(kh.at[dstall.at[0]], kbuf.at[b],
                              semg[b]).wait()
        pltpu.make_async_copy(qvh.at[srcb.at[b]], qvbuf.at[b],
                              semg[b]).wait()

    def wait_scatter(b):
        pltpu.make_async_copy(msgb.at[b], agg.at[dstall.at[0]],
                              sems[b]).wait()

    def do_round(r, b):
        wait_gather(b)
        @pl.when(r + 1 < _R)
        def _prefetch():
            wait_idx(1 - b)
            issue_gather(r + 1, 1 - b)
        @pl.when(r >= 1)
        def _drain():
            wait_scatter(1 - b)

        unpack = functools.partial(plsc.unpack,
                                   format=plsc.PackFormat.INTERLEAVED,
                                   preferred_element_type=jnp.float32)

        @plsc.parallel_loop(0, _G, step=_UNROLL, unroll=2)
        def edge_block(i):
            for jj in range(_UNROLL):
                j = i + jj
                eaj = eab[b, j]
                for gp in range(_NCH // 2):
                    k0, k1 = unpack(kbuf[b, j, pl.ds(32 * gp, 32)])
                    q0, q1 = unpack(qvbuf[b, j, pl.ds(32 * gp, 32)])
                    v0, v1 = unpack(qvbuf[b, j, pl.ds(_HH + 32 * gp, 32)])
                    for kc, qc, vc, ci in ((k0, q0, v0, 2 * gp),
                                           (k1, q1, v1, 2 * gp + 1)):
                        t = kc + qc + eaj * ewc[ci]
                        gate = 1.0 / (1.0 + jnp.exp(-t))
                        msgb[b, j, pl.ds(ci * 16, 16)] = gate * vc

        pltpu.async_copy(msgb.at[b], agg.at[dstall.at[r]], sems[b], add=True)
        @pl.when(r + 2 < _R)
        def _idx_prefetch():
            issue_idx(r + 2, b)

    issue_idx(0, 0)
    wait_idx(0)
    issue_gather(0, 0)
    issue_idx(1, 1)

    def pair_body(rr, carry):
        do_round(rr * 2, 0)
        do_round(rr * 2 + 1, 1)
        return carry

    lax.fori_loop(0, _R // 2, pair_body, 0)
    if _R % 2:
        do_round(_R - 1, 0)
        wait_scatter(0)
    else:
        wait_scatter(1)
    plsc.subcore_barrier()
    pltpu.sync_copy(agg.at[pl.ds(row0, _NPS)],
                    out_hbm.at[cid, pl.ds(row0, _NPS)])
    @pl.when(sid == 0)
    def _out_tail():
        pltpu.sync_copy(agg.at[pl.ds(_NS * _NPS, _NTAIL)],
                        out_hbm.at[cid, pl.ds(_NS * _NPS, _NTAIL)])


@functools.lru_cache(maxsize=1)
def _sc_edge_kernel():
    return pl.kernel(
        _sc_edge_body,
        mesh=plsc.VectorSubcoreMesh(core_axis_name="c", subcore_axis_name="s"),
        compiler_params=pltpu.CompilerParams(use_tc_tiling_on_sc=False,
                                             needs_layout_passes=False),
        out_type=jax.ShapeDtypeStruct((2, _N, _HH), jnp.float32),
        scratch_types=[
            pltpu.VMEM((_R, _G), jnp.int32),
            pltpu.VMEM((2, _G), jnp.int32),
            pltpu.VMEM((2, _G, 16), jnp.float32),
            pltpu.VMEM((_HH,), jnp.float32),
            pltpu.VMEM((2, _G, _HH), jnp.bfloat16),
            pltpu.VMEM((2, _G, 2 * _HH), jnp.bfloat16),
            pltpu.VMEM((2, _G, _HH), jnp.float32),
            pltpu.VMEM_SHARED((_N, _HH), jnp.float32),
            pltpu.SemaphoreType.DMA,
            pltpu.SemaphoreType.DMA,
            pltpu.SemaphoreType.DMA,
            pltpu.SemaphoreType.DMA,
            pltpu.SemaphoreType.DMA,
            pltpu.SemaphoreType.DMA,
        ],
    )


def _sc_edge(*args):
    return _sc_edge_kernel()(*args)


def _feature_perm():
    idx = []
    for c in range(2):
        for gp in range(2):
            base = c * 64 + 32 * gp
            idx += [base + 2 * i for i in range(16)]
            idx += [base + 2 * i + 1 for i in range(16)]
    return idx


_P = np.array(_feature_perm())


def kernel(x, edge_index, edge_attr, params):
    src = edge_index[0].reshape(_NS, _R, _G)
    dst = edge_index[1].reshape(_NS, _R, _G)
    edge_attr = jnp.broadcast_to(
        edge_attr.reshape(_NS, _R, _G, 1), (_NS, _R, _G, 16))
    zeros = jnp.zeros((_N, _HH), jnp.float32)
    blocks = []
    for i, blk in enumerate(params['blocks']):
        kw, qw, vw, sw = blk['kw'], blk['qw'], blk['vw'], blk['sw']
        if i > 0:
            kw, qw, vw, sw = kw[_P, :], qw[_P, :], vw[_P, :], sw[_P, :]
        blocks.append(dict(
            kw=kw, kb=blk['kb'], eb=blk['eb'],
            qw=qw, qb=blk['qb'],
            vw=vw, vb=blk['vb'],
            sw=sw[:, _P], bias=blk['bias'][_P],
            g=blk['g'][_P], be=blk['be'][_P],
            ew=blk['ew'][:, _P],
        ))
    out_w = params['out_w'][_P, :]
    k, qv, skip = _tc_first(x, params['in_w'], params['in_b'], blocks[0])
    y = None
    for i in range(len(blocks)):
        blk = blocks[i]
        ew2 = blk['ew'][0].reshape(2, _HH)
        agg = _sc_edge(k, qv, src, dst, edge_attr, ew2, zeros)
        if i + 1 < len(blocks):
            k, qv, skip = _tc_mid(agg, skip, blk['g'], blk['be'],
                                  blocks[i + 1])
        else:
            y = _tc_final(agg, skip, blk['g'], blk['be'], out_w)
    return y


# speedup vs baseline: 3.2021x; 1.2090x over previous
import functools

import numpy as np

import jax
import jax.numpy as jnp
from jax import lax
from jax.experimental import pallas as pl
from jax.experimental.pallas import tpu as pltpu
from jax.experimental.pallas import tpu_sc as plsc

_N = 10000
_E = 320000
_H = 128
_HH = _H // 2

_NC = 2
_NS = 16
_EPT = _E // _NS
_G = 80
_R = _EPT // _G
_UNROLL = 4
_NCH = _HH // 16
_NPS = 624
_NTAIL = _N - _NS * _NPS

_BR = 2000
_GRID = _N // _BR


def _proj(h, w_ref, b):
    return jnp.dot(h, w_ref[...], preferred_element_type=jnp.float32) + b


def _kqvs(h, kw, kb, eb, qw, qb, vw, vb, sw, sb, k_out, qv_out, skip_out):
    k = _proj(h, kw, kb[...] + eb[...]).astype(jnp.bfloat16)
    q = _proj(h, qw, qb[...]).astype(jnp.bfloat16)
    v = _proj(h, vw, vb[...]).astype(jnp.bfloat16)
    k_out[0] = k[:, :_HH]
    k_out[1] = k[:, _HH:]
    qv_out[0] = jnp.concatenate([q[:, :_HH], v[:, :_HH]], axis=1)
    qv_out[1] = jnp.concatenate([q[:, _HH:], v[:, _HH:]], axis=1)
    skip_out[...] = _proj(h, sw, sb[...])


def _tc_first_body(x_ref, in_w, in_b, kw, kb, eb, qw, qb, vw, vb, sw, sb,
                   k_out, qv_out, skip_out):
    h = _proj(x_ref[...], in_w, in_b[...])
    _kqvs(h, kw, kb, eb, qw, qb, vw, vb, sw, sb, k_out, qv_out, skip_out)


def _ln_gelu(agg_ref, skip_ref, g, be):
    out = jnp.concatenate([agg_ref[0], agg_ref[1]], axis=1) + skip_ref[...]
    mu = jnp.mean(out, axis=-1, keepdims=True)
    var = jnp.mean((out - mu) ** 2, axis=-1, keepdims=True)
    xn = (out - mu) * lax.rsqrt(var + 1e-5) * g[...] + be[...]
    return 0.5 * xn * (1.0 + lax.erf(xn * (2.0 ** -0.5)))


def _tc_mid_body(agg_ref, skip_ref, g, be, kw, kb, eb, qw, qb, vw, vb, sw, sb,
                 k_out, qv_out, skip_out):
    h = _ln_gelu(agg_ref, skip_ref, g, be)
    _kqvs(h, kw, kb, eb, qw, qb, vw, vb, sw, sb, k_out, qv_out, skip_out)


def _tc_final_body(agg_ref, skip_ref, g, be, ow, y_out):
    h = _ln_gelu(agg_ref, skip_ref, g, be)
    y_out[...] = jnp.dot(h, ow[...], preferred_element_type=jnp.float32)


def _wspec(shape):
    nd = len(shape)
    return pl.BlockSpec(shape, lambda i, _n=nd: (0,) * _n)


_ROW = pl.BlockSpec((_BR, _H), lambda i: (i, 0))
_HALF = pl.BlockSpec((2, _BR, _HH), lambda i: (0, i, 0))
_HALF2 = pl.BlockSpec((2, _BR, _H), lambda i: (0, i, 0))

_KQVS_OUT = dict(
    out_specs=[_HALF, _HALF2, _ROW],
    out_shape=[
        jax.ShapeDtypeStruct((2, _N, _HH), jnp.bfloat16),
        jax.ShapeDtypeStruct((2, _N, _H), jnp.bfloat16),
        jax.ShapeDtypeStruct((_N, _H), jnp.float32),
    ],
)

_W = _wspec((_H, _H))
_B = _wspec((1, _H))


def _tc_first(x, in_w, in_b, blk):
    f = pl.pallas_call(
        _tc_first_body,
        grid=(_GRID,),
        in_specs=[_ROW, _W, _B] + [_W, _B, _B, _W, _B, _W, _B, _W, _B],
        **_KQVS_OUT,
    )
    return f(x, in_w, in_b.reshape(1, -1),
             blk['kw'], blk['kb'].reshape(1, -1), blk['eb'].reshape(1, -1),
             blk['qw'], blk['qb'].reshape(1, -1),
             blk['vw'], blk['vb'].reshape(1, -1),
             blk['sw'], blk['bias'].reshape(1, -1))


def _tc_mid(agg, skip, g, be, blk):
    f = pl.pallas_call(
        _tc_mid_body,
        grid=(_GRID,),
        in_specs=[_HALF, _ROW, _B, _B] + [_W, _B, _B, _W, _B, _W, _B, _W, _B],
        **_KQVS_OUT,
    )
    return f(agg, skip, g.reshape(1, -1), be.reshape(1, -1),
             blk['kw'], blk['kb'].reshape(1, -1), blk['eb'].reshape(1, -1),
             blk['qw'], blk['qb'].reshape(1, -1),
             blk['vw'], blk['vb'].reshape(1, -1),
             blk['sw'], blk['bias'].reshape(1, -1))


def _tc_final(agg, skip, g, be, ow):
    f = pl.pallas_call(
        _tc_final_body,
        grid=(_GRID,),
        in_specs=[_HALF, _ROW, _B, _B, _W],
        out_specs=_ROW,
        out_shape=jax.ShapeDtypeStruct((_N, _H), jnp.float32),
    )
    return f(agg, skip, g.reshape(1, -1), be.reshape(1, -1), ow)


def _sc_edge_body(k_hbm, qv_hbm, src_hbm, dst_hbm, ea_hbm, ew_hbm, zero_hbm,
                  out_hbm, dstall, srcall, eab, ewb, kbuf, qvbuf, msgb, agg,
                  semg0, semg1, sems0, sems1, semi0, semi1):
    semg = (semg0, semg1)
    sems = (sems0, sems1)
    semi = (semi0, semi1)
    cid = lax.axis_index("c")
    sid = lax.axis_index("s")
    row0 = sid * _NPS
    kh = k_hbm.at[cid]
    qvh = qv_hbm.at[cid]
    pltpu.sync_copy(zero_hbm.at[pl.ds(row0, _NPS)], agg.at[pl.ds(row0, _NPS)])
    @pl.when(sid == 0)
    def _init_tail():
        pltpu.sync_copy(zero_hbm.at[pl.ds(_NS * _NPS, _NTAIL)],
                        agg.at[pl.ds(_NS * _NPS, _NTAIL)])
    pltpu.sync_copy(ew_hbm.at[cid], ewb)
    pltpu.sync_copy(dst_hbm.at[sid], dstall)
    pltpu.sync_copy(src_hbm.at[sid], srcall)
    plsc.subcore_barrier()

    ewc = [ewb[pl.ds(c * 16, 16)] for c in range(_NCH)]

    def issue_idx(r, b):
        pltpu.async_copy(ea_hbm.at[sid, r], eab.at[b], semi[b])

    def wait_idx(b):
        pltpu.make_async_copy(ea_hbm.at[sid, 0], eab.at[b], semi[b]).wait()

    def issue_gather(r, b):
        pltpu.async_copy(kh.at[dstall.at[r]], kbuf.at[b], semg[b])
        pltpu.async_copy(qvh.at[srcall.at[r]], qvbuf.at[b], semg[b])

    def wait_gather(b):
        pltpu.make_async_copy(kh.at[dstall.at[0]], kbuf.at[b],
                              semg[b]).wait()
        pltpu.make_async_copy(qvh.at[srcall.at[0]], qvbuf.at[b],
                              semg[b]).wait()

    def wait_scatter(b):
        pltpu.make_async_copy(msgb.at[b], agg.at[dstall.at[0]],
                              sems[b]).wait()

    def do_round(r, b):
        @pl.when(r + 1 < _R)
        def _prefetch():
            issue_gather(r + 1, 1 - b)
        wait_gather(b)
        @pl.when(r >= 1)
        def _drain():
            wait_scatter(1 - b)
        wait_idx(b)

        unpack = functools.partial(plsc.unpack,
                                   format=plsc.PackFormat.INTERLEAVED,
                                   preferred_element_type=jnp.float32)

        @plsc.parallel_loop(0, _G, step=_UNROLL, unroll=2)
        def edge_block(i):
            for jj in range(_UNROLL):
                j = i + jj
                eaj = eab[b, j]
                for gp in range(_NCH // 2):
                    k0, k1 = unpack(kbuf[b, j, pl.ds(32 * gp, 32)])
                    q0, q1 = unpack(qvbuf[b, j, pl.ds(32 * gp, 32)])
                    v0, v1 = unpack(qvbuf[b, j, pl.ds(_HH + 32 * gp, 32)])
                    for kc, qc, vc, ci in ((k0, q0, v0, 2 * gp),
                                           (k1, q1, v1, 2 * gp + 1)):
                        t = kc + qc + eaj * ewc[ci]
                        gate = 1.0 / (1.0 + jnp.exp(-t))
                        msgb[b, j, pl.ds(ci * 16, 16)] = gate * vc

        pltpu.async_copy(msgb.at[b], agg.at[dstall.at[r]], sems[b], add=True)
        @pl.when(r + 2 < _R)
        def _idx_prefetch():
            issue_idx(r + 2, b)

    issue_idx(0, 0)
    issue_idx(1, 1)
    issue_gather(0, 0)

    def pair_body(rr, carry):
        do_round(rr * 2, 0)
        do_round(rr * 2 + 1, 1)
        return carry

    lax.fori_loop(0, _R // 2, pair_body, 0)
    if _R % 2:
        do_round(_R - 1, 0)
        wait_scatter(0)
    else:
        wait_scatter(1)
    plsc.subcore_barrier()
    pltpu.sync_copy(agg.at[pl.ds(row0, _NPS)],
                    out_hbm.at[cid, pl.ds(row0, _NPS)])
    @pl.when(sid == 0)
    def _out_tail():
        pltpu.sync_copy(agg.at[pl.ds(_NS * _NPS, _NTAIL)],
                        out_hbm.at[cid, pl.ds(_NS * _NPS, _NTAIL)])


@functools.lru_cache(maxsize=1)
def _sc_edge_kernel():
    return pl.kernel(
        _sc_edge_body,
        mesh=plsc.VectorSubcoreMesh(core_axis_name="c", subcore_axis_name="s"),
        compiler_params=pltpu.CompilerParams(use_tc_tiling_on_sc=False,
                                             needs_layout_passes=False),
        out_type=jax.ShapeDtypeStruct((2, _N, _HH), jnp.float32),
        scratch_types=[
            pltpu.VMEM((_R, _G), jnp.int32),
            pltpu.VMEM((_R, _G), jnp.int32),
            pltpu.VMEM((2, _G, 16), jnp.float32),
            pltpu.VMEM((_HH,), jnp.float32),
            pltpu.VMEM((2, _G, _HH), jnp.bfloat16),
            pltpu.VMEM((2, _G, 2 * _HH), jnp.bfloat16),
            pltpu.VMEM((2, _G, _HH), jnp.float32),
            pltpu.VMEM_SHARED((_N, _HH), jnp.float32),
            pltpu.SemaphoreType.DMA,
            pltpu.SemaphoreType.DMA,
            pltpu.SemaphoreType.DMA,
            pltpu.SemaphoreType.DMA,
            pltpu.SemaphoreType.DMA,
            pltpu.SemaphoreType.DMA,
        ],
    )


def _sc_edge(*args):
    return _sc_edge_kernel()(*args)


def _feature_perm():
    idx = []
    for c in range(2):
        for gp in range(2):
            base = c * 64 + 32 * gp
            idx += [base + 2 * i for i in range(16)]
            idx += [base + 2 * i + 1 for i in range(16)]
    return idx


_P = np.array(_feature_perm())


def kernel(x, edge_index, edge_attr, params):
    src = edge_index[0].reshape(_NS, _R, _G)
    dst = edge_index[1].reshape(_NS, _R, _G)
    edge_attr = jnp.broadcast_to(
        edge_attr.reshape(_NS, _R, _G, 1), (_NS, _R, _G, 16))
    zeros = jnp.zeros((_N, _HH), jnp.float32)
    blocks = []
    for i, blk in enumerate(params['blocks']):
        kw, qw, vw, sw = blk['kw'], blk['qw'], blk['vw'], blk['sw']
        if i > 0:
            kw, qw, vw, sw = kw[_P, :], qw[_P, :], vw[_P, :], sw[_P, :]
        blocks.append(dict(
            kw=kw, kb=blk['kb'], eb=blk['eb'],
            qw=qw, qb=blk['qb'],
            vw=vw, vb=blk['vb'],
            sw=sw[:, _P], bias=blk['bias'][_P],
            g=blk['g'][_P], be=blk['be'][_P],
            ew=blk['ew'][:, _P],
        ))
    out_w = params['out_w'][_P, :]
    k, qv, skip = _tc_first(x, params['in_w'], params['in_b'], blocks[0])
    y = None
    for i in range(len(blocks)):
        blk = blocks[i]
        ew2 = blk['ew'][0].reshape(2, _HH)
        agg = _sc_edge(k, qv, src, dst, edge_attr, ew2, zeros)
        if i + 1 < len(blocks):
            k, qv, skip = _tc_mid(agg, skip, blk['g'], blk['be'],
                                  blocks[i + 1])
        else:
            y = _tc_final(agg, skip, blk['g'], blk['be'], out_w)
    return y
